# speedup vs baseline: 1.6436x; 1.6436x over previous
import functools

import jax
import jax.numpy as jnp
from jax import lax
from jax.experimental import pallas as pl
from jax.experimental.pallas import tpu as pltpu

NE = 8
TOPK = 2
H = 1024
F = 2048
T = 2048
TB = 512


def _router_body(x_ref, gw_ref, w8_ref):
    x = x_ref[...]
    logits = lax.dot_general(
        x, gw_ref[...], (((1,), (1,)), ((), ())),
        preferred_element_type=jnp.float32)
    m = jnp.max(logits, axis=1, keepdims=True)
    e = jnp.exp(logits - m)
    p = e / jnp.sum(e, axis=1, keepdims=True)

    ii = lax.broadcasted_iota(jnp.int32, (T, NE), 1)
    big = jnp.int32(NE)
    pm1 = jnp.max(p, axis=1, keepdims=True)
    idx1 = jnp.min(jnp.where(p == pm1, ii, big), axis=1, keepdims=True)
    mask1 = ii == idx1
    p2 = jnp.where(mask1, -1.0, p)
    pm2 = jnp.max(p2, axis=1, keepdims=True)
    idx2 = jnp.min(jnp.where(p2 == pm2, ii, big), axis=1, keepdims=True)
    mask2 = ii == idx2
    w8 = jnp.where(mask1 | mask2, p, 0.0) / (pm1 + pm2)
    w8_ref[...] = w8


def _expert_body(x_ref, w8_ref, gpw_ref, upw_ref, dpw_ref, out_ref):
    e = pl.program_id(0)
    f = pl.program_id(1)

    @pl.when(jnp.logical_and(e == 0, f == 0))
    def _():
        out_ref[...] = jnp.zeros_like(out_ref)

    gw = gpw_ref[0].astype(jnp.bfloat16)
    uw = upw_ref[0].astype(jnp.bfloat16)
    dw = dpw_ref[0].astype(jnp.bfloat16)

    lane = lax.broadcasted_iota(jnp.int32, (TB, NE), 1)
    for b in range(T // TB):
        sl = pl.ds(b * TB, TB)
        xb = x_ref[sl, :].astype(jnp.bfloat16)
        g = lax.dot_general(xb, gw, (((1,), (1,)), ((), ())),
                            preferred_element_type=jnp.float32)
        u = lax.dot_general(xb, uw, (((1,), (1,)), ((), ())),
                            preferred_element_type=jnp.float32)
        h = (g * (1.0 / (1.0 + jnp.exp(-g))) * u).astype(jnp.bfloat16)
        o = lax.dot_general(h, dw, (((1,), (1,)), ((), ())),
                            preferred_element_type=jnp.float32)
        w8b = w8_ref[sl, :]
        wcol = jnp.sum(jnp.where(lane == e, w8b, 0.0), axis=1, keepdims=True)
        out_ref[sl, :] += o * wcol


@jax.jit
def kernel(x, gate_w, gate_proj_w, up_proj_w, down_proj_w):
    Bs, Ss, Hh = x.shape
    x_flat = x.reshape(T, H)

    w8 = pl.pallas_call(
        _router_body,
        out_shape=jax.ShapeDtypeStruct((T, NE), jnp.float32),
    )(x_flat, gate_w)

    out = pl.pallas_call(
        _expert_body,
        grid=(NE, 2),
        in_specs=[
            pl.BlockSpec((T, H), lambda e, f: (0, 0)),
            pl.BlockSpec((T, NE), lambda e, f: (0, 0)),
            pl.BlockSpec((1, H, H), lambda e, f: (e, f, 0)),
            pl.BlockSpec((1, H, H), lambda e, f: (e, f, 0)),
            pl.BlockSpec((1, H, H), lambda e, f: (e, 0, f)),
        ],
        out_specs=pl.BlockSpec((T, H), lambda e, f: (0, 0)),
        out_shape=jax.ShapeDtypeStruct((T, H), jnp.float32),
        compiler_params=pltpu.CompilerParams(
            dimension_semantics=("arbitrary", "arbitrary"),
        ),
    )(x_flat, w8, gate_proj_w, up_proj_w, down_proj_w)

    return out.reshape(Bs, Ss, Hh)
